# baseline (speedup 1.0000x reference)
import jax
import jax.numpy as jnp
from jax import lax
from jax.experimental import pallas as pl
from jax.experimental.pallas import tpu as pltpu
from jax.experimental.pallas import tpu_sc as plsc

BATCH = 16384
D = 32
NC = 2
NS = 16
NW = NC * NS
BPW = BATCH // NW
CHUNK = 128
NCH = BPW // CHUNK
GROUPS = BPW // 16


def _sc_body(uid_hbm, iid_hbm, uemb_hbm, iemb_hbm, ubias_hbm, ibias_hbm,
             out_hbm,
             uid_v, iid_v, urows_v, irows_v, ub_v, ib_v, out_v, sem):
    wid = lax.axis_index("s") * NC + lax.axis_index("c")
    base = wid * BPW

    for j in range(NCH):
        pltpu.sync_copy(uid_hbm.at[pl.ds(base + j * CHUNK, CHUNK)], uid_v.at[j])
        pltpu.sync_copy(iid_hbm.at[pl.ds(base + j * CHUNK, CHUNK)], iid_v.at[j])

    copies = []
    for j in range(NCH):
        sl = pl.ds(j * CHUNK, CHUNK)
        copies.append(pltpu.async_copy(uemb_hbm.at[uid_v.at[j]], urows_v.at[sl], sem))
        copies.append(pltpu.async_copy(iemb_hbm.at[iid_v.at[j]], irows_v.at[sl], sem))
        copies.append(pltpu.async_copy(ubias_hbm.at[uid_v.at[j]], ub_v.at[sl], sem))
        copies.append(pltpu.async_copy(ibias_hbm.at[iid_v.at[j]], ib_v.at[sl], sem))
    for c in copies:
        c.wait()

    lane = lax.iota(jnp.int32, 16)

    def g_body(g, carry):
        rows = lane + g * 16
        acc = ub_v[pl.ds(g * 16, 16)] + ib_v[pl.ds(g * 16, 16)]
        for d in range(D):
            col = jnp.full((16,), d, jnp.int32)
            u = plsc.load_gather(urows_v, [rows, col])
            it = plsc.load_gather(irows_v, [rows, col])
            acc = acc + u * it
        out_v[pl.ds(g * 16, 16)] = acc
        return carry

    lax.fori_loop(0, GROUPS, g_body, 0)
    pltpu.sync_copy(out_v, out_hbm.at[pl.ds(base, BPW)])


def kernel(user_ids, item_ids, user_embeddings, item_embeddings,
           user_bias, item_bias, global_bias):
    mesh = plsc.VectorSubcoreMesh(core_axis_name="c", subcore_axis_name="s")
    k = pl.kernel(
        _sc_body,
        out_type=jax.ShapeDtypeStruct((BATCH,), jnp.float32),
        mesh=mesh,
        compiler_params=pltpu.CompilerParams(
            needs_layout_passes=False, use_tc_tiling_on_sc=False),
        scratch_types=[
            pltpu.VMEM((NCH, CHUNK), jnp.int32),
            pltpu.VMEM((NCH, CHUNK), jnp.int32),
            pltpu.VMEM((BPW, D), jnp.float32),
            pltpu.VMEM((BPW, D), jnp.float32),
            pltpu.VMEM((BPW,), jnp.float32),
            pltpu.VMEM((BPW,), jnp.float32),
            pltpu.VMEM((BPW,), jnp.float32),
            pltpu.SemaphoreType.DMA,
        ],
    )
    out = k(user_ids.astype(jnp.int32), item_ids.astype(jnp.int32),
            user_embeddings, item_embeddings, user_bias, item_bias)
    return out + global_bias
